# Initial kernel scaffold; baseline (speedup 1.0000x reference)
import functools

import jax
import jax.numpy as jnp
from jax import lax
from jax.experimental import pallas as pl
from jax.experimental.pallas import tpu as pltpu
from jax.experimental.pallas import tpu_sc as plsc

T = 2048
D = 768
E = 8
F = 3072
K = 2
A = T * K
BLK = 256
NB = A // BLK + E
NPAD = NB * BLK
PCH = 512

NW = 32
GCH = 96
CCH = 64


def _router_body(x_ref, wg_ref, bg_ref, pos_ref, tok_ref, gate_ref, be_ref):
    xf = x_ref[...]
    logits = lax.dot_general(
        wg_ref[...], xf, (((0,), (1,)), ((), ())),
        preferred_element_type=jnp.float32) + bg_ref[...]
    m = jnp.max(logits, axis=0, keepdims=True)
    ex = jnp.exp(logits - m)
    gates = ex / jnp.sum(ex, axis=0, keepdims=True)

    erow = lax.broadcasted_iota(jnp.float32, (E, T), 0)
    g1 = jnp.max(gates, axis=0, keepdims=True)
    i1 = jnp.min(jnp.where(gates == g1, erow, jnp.float32(E)),
                 axis=0, keepdims=True)
    gates2 = jnp.where(erow == i1, -jnp.inf, gates)
    g2 = jnp.max(gates2, axis=0, keepdims=True)
    i2 = jnp.min(jnp.where(gates2 == g2, erow, jnp.float32(E)),
                 axis=0, keepdims=True)

    ea = jnp.concatenate([i1, i2], axis=1)
    gj = jnp.concatenate([g1, g2], axis=1)

    erowA = lax.broadcasted_iota(jnp.float32, (E, A), 0)
    oh = (ea == erowA).astype(jnp.float32)
    cs = oh
    sh = 1
    while sh < A:
        cs = cs + jnp.concatenate(
            [jnp.zeros((E, sh), jnp.float32), cs[:, :-sh]], axis=1)
        sh *= 2
    counts = cs[:, A - 1:A]
    nblk = jnp.floor((counts + (BLK - 1)) * (1.0 / BLK))
    padded = nblk * BLK
    ltri = (lax.broadcasted_iota(jnp.float32, (E, E), 0)
            > lax.broadcasted_iota(jnp.float32, (E, E), 1)).astype(jnp.float32)
    off = jnp.dot(ltri, padded, preferred_element_type=jnp.float32)
    ends_blk = (off + padded) * (1.0 / BLK)

    posf = jnp.sum(oh * (off + cs - 1.0), axis=0, keepdims=True)
    pos_ref[...] = posf.astype(jnp.int32)

    bcol = lax.broadcasted_iota(jnp.float32, (1, NB), 1)
    be = jnp.sum((bcol >= ends_blk).astype(jnp.float32), axis=0, keepdims=True)
    be_ref[...] = jnp.minimum(be, jnp.float32(E - 1)).astype(jnp.int32)

    tokj = jnp.concatenate(
        [lax.broadcasted_iota(jnp.float32, (1, T), 1)] * 2, axis=1)
    for c in range(NPAD // PCH):
        pcol = lax.broadcasted_iota(jnp.float32, (PCH, 1), 0) + (c * PCH)
        mask = (posf == pcol).astype(jnp.float32)
        tok_ref[c * PCH:(c + 1) * PCH, :] = (
            jnp.sum(mask * tokj, axis=1, keepdims=True).astype(jnp.int32))
        gate_ref[c * PCH:(c + 1) * PCH, :] = (
            jnp.sum(mask * gj, axis=1, keepdims=True))


def _router_call(xf, Wg, bg):
    return pl.pallas_call(
        _router_body,
        out_shape=(
            jax.ShapeDtypeStruct((1, A), jnp.int32),
            jax.ShapeDtypeStruct((NPAD, 1), jnp.int32),
            jax.ShapeDtypeStruct((NPAD, 1), jnp.float32),
            jax.ShapeDtypeStruct((1, NB), jnp.int32),
        ),
    )(xf, Wg, bg.reshape(1, E))


_SC_MESH = plsc.VectorSubcoreMesh(core_axis_name="c", subcore_axis_name="s")


@functools.partial(
    pl.kernel,
    mesh=_SC_MESH,
    out_type=jax.ShapeDtypeStruct((NPAD, D), jnp.float32),
    scratch_types=[
        pltpu.VMEM((GCH,), jnp.int32),
        pltpu.VMEM((GCH, D), jnp.float32),
        pltpu.SemaphoreType.DMA,
    ],
)
def _sc_dispatch(x_hbm, tok_hbm, out_hbm, idx_v, rows_v, sem):
    wid = lax.axis_index("s") * 2 + lax.axis_index("c")
    rows_per_w = NPAD // NW
    for c in range(rows_per_w // GCH):
        base = wid * rows_per_w + c * GCH
        pltpu.sync_copy(tok_hbm.at[pl.ds(base, GCH)], idx_v)
        pltpu.async_copy(x_hbm.at[idx_v], rows_v, sem).wait()
        pltpu.sync_copy(rows_v, out_hbm.at[pl.ds(base, GCH)])


_SQRT_HALF = 0.7071067811865476


def _gemm_body(be_ref, x_ref, w1_ref, b1_ref, w2_ref, b2_ref, g_ref, o_ref):
    xb = x_ref[...]
    h = jnp.dot(xb, w1_ref[0], preferred_element_type=jnp.float32) + b1_ref[0]
    h = 0.5 * h * (1.0 + lax.erf(h * _SQRT_HALF))
    o = jnp.dot(h, w2_ref[0], preferred_element_type=jnp.float32) + b2_ref[0]
    o_ref[...] = o * g_ref[0, 0][:, None]


def _gemm_call(blk_expert, x_sorted, W1, b1, W2, b2, gate_sorted):
    grid_spec = pltpu.PrefetchScalarGridSpec(
        num_scalar_prefetch=1,
        grid=(NB,),
        in_specs=[
            pl.BlockSpec((BLK, D), lambda i, s: (i, 0)),
            pl.BlockSpec((1, D, F), lambda i, s: (s[i], 0, 0)),
            pl.BlockSpec((1, 1, F), lambda i, s: (s[i], 0, 0)),
            pl.BlockSpec((1, F, D), lambda i, s: (s[i], 0, 0)),
            pl.BlockSpec((1, 1, D), lambda i, s: (s[i], 0, 0)),
            pl.BlockSpec((1, 1, BLK), lambda i, s: (i, 0, 0)),
        ],
        out_specs=pl.BlockSpec((BLK, D), lambda i, s: (i, 0)),
    )
    return pl.pallas_call(
        _gemm_body,
        grid_spec=grid_spec,
        out_shape=jax.ShapeDtypeStruct((NPAD, D), jnp.float32),
    )(blk_expert, x_sorted, W1, b1.reshape(E, 1, F), W2, b2.reshape(E, 1, D),
      gate_sorted.reshape(NB, 1, BLK))


@functools.partial(
    pl.kernel,
    mesh=_SC_MESH,
    out_type=jax.ShapeDtypeStruct((T, D), jnp.float32),
    scratch_types=[
        pltpu.VMEM((CCH,), jnp.int32),
        pltpu.VMEM((CCH, D), jnp.float32),
        pltpu.VMEM((CCH, D), jnp.float32),
        pltpu.SemaphoreType.DMA,
        pltpu.SemaphoreType.DMA,
    ],
)
def _sc_combine(h_hbm, pos1_hbm, pos2_hbm, out_hbm, idx_v, a_v, b_v, s1, s2):
    wid = lax.axis_index("s") * 2 + lax.axis_index("c")
    base = wid * CCH
    pltpu.sync_copy(pos1_hbm.at[pl.ds(base, CCH)], idx_v)
    pltpu.async_copy(h_hbm.at[idx_v], a_v, s1).wait()
    pltpu.sync_copy(pos2_hbm.at[pl.ds(base, CCH)], idx_v)
    pltpu.async_copy(h_hbm.at[idx_v], b_v, s2).wait()

    def row_add(i, carry):
        for k in range(D // 16):
            sl = pl.ds(k * 16, 16)
            a_v[i, sl] = a_v[i, sl] + b_v[i, sl]
        return carry

    lax.fori_loop(0, CCH, row_add, 0)
    pltpu.sync_copy(a_v, out_hbm.at[pl.ds(base, CCH)])


def kernel(x, Wg, bg, W1, b1, W2, b2):
    b, t, d = x.shape
    xf = x.reshape(T, D)
    pos, tok_sorted, gate_sorted, blk_expert = _router_call(xf, Wg, bg)
    x_sorted = _sc_dispatch(xf, tok_sorted.reshape(NPAD))
    H = _gemm_call(blk_expert.reshape(NB), x_sorted, W1, b1, W2, b2,
                   gate_sorted.reshape(NPAD))
    pos2d = pos.reshape(K, T)
    out = _sc_combine(H, pos2d[0], pos2d[1])
    return out.reshape(b, t, d)


# speedup vs baseline: 2.2213x; 2.2213x over previous
import functools

import jax
import jax.numpy as jnp
from jax import lax
from jax.experimental import pallas as pl
from jax.experimental.pallas import tpu as pltpu
from jax.experimental.pallas import tpu_sc as plsc

T = 2048
D = 768
E = 8
F = 3072
K = 2
A = T * K
BLK = 256
NB = A // BLK + E
NPAD = NB * BLK
PCH = 512

NW = 32
GCH = 96
CCH = 64


def _fiota(shape, dim):
    return lax.broadcasted_iota(jnp.int32, shape, dim).astype(jnp.float32)


def _router_body(x_ref, wg_ref, bg_ref, pos_ref, tok_ref, gate_ref, be_ref):
    xf = x_ref[...]
    logits = lax.dot_general(
        wg_ref[...], xf, (((0,), (1,)), ((), ())),
        preferred_element_type=jnp.float32) + bg_ref[...]
    m = jnp.max(logits, axis=0, keepdims=True)
    ex = jnp.exp(logits - m)
    gates = ex / jnp.sum(ex, axis=0, keepdims=True)

    erow = _fiota( (E, T), 0)
    g1 = jnp.max(gates, axis=0, keepdims=True)
    i1 = jnp.min(jnp.where(gates == g1, erow, jnp.float32(E)),
                 axis=0, keepdims=True)
    gates2 = jnp.where(erow == i1, -jnp.inf, gates)
    g2 = jnp.max(gates2, axis=0, keepdims=True)
    i2 = jnp.min(jnp.where(gates2 == g2, erow, jnp.float32(E)),
                 axis=0, keepdims=True)

    ea = jnp.concatenate([i1, i2], axis=1)
    gj = jnp.concatenate([g1, g2], axis=1)

    erowA = _fiota( (E, A), 0)
    oh = (ea == erowA).astype(jnp.float32)
    cs = oh
    sh = 1
    while sh < A:
        cs = cs + jnp.concatenate(
            [jnp.zeros((E, sh), jnp.float32), cs[:, :-sh]], axis=1)
        sh *= 2
    counts = cs[:, A - 1:A]
    nblk = jnp.floor((counts + (BLK - 1)) * (1.0 / BLK))
    padded = nblk * BLK
    ltri = (_fiota( (E, E), 0)
            > _fiota( (E, E), 1)).astype(jnp.float32)
    off = jnp.dot(ltri, padded, preferred_element_type=jnp.float32)
    ends_blk = (off + padded) * (1.0 / BLK)

    posf = jnp.sum(oh * (off + cs - 1.0), axis=0, keepdims=True)
    pos_ref[...] = posf.astype(jnp.int32)

    bcol = _fiota( (1, NB), 1)
    be = jnp.sum((bcol >= ends_blk).astype(jnp.float32), axis=0, keepdims=True)
    be_ref[...] = jnp.minimum(be, jnp.float32(E - 1)).astype(jnp.int32)

    tokj = jnp.concatenate(
        [_fiota( (1, T), 1)] * 2, axis=1)
    for c in range(NPAD // PCH):
        pcol = _fiota( (PCH, 1), 0) + (c * PCH)
        mask = (posf == pcol).astype(jnp.float32)
        tok_ref[c * PCH:(c + 1) * PCH, :] = (
            jnp.sum(mask * tokj, axis=1, keepdims=True).astype(jnp.int32))
        gate_ref[c * PCH:(c + 1) * PCH, :] = (
            jnp.sum(mask * gj, axis=1, keepdims=True))


def _router_call(xf, Wg, bg):
    return pl.pallas_call(
        _router_body,
        out_shape=(
            jax.ShapeDtypeStruct((1, A), jnp.int32),
            jax.ShapeDtypeStruct((NPAD, 1), jnp.int32),
            jax.ShapeDtypeStruct((NPAD, 1), jnp.float32),
            jax.ShapeDtypeStruct((1, NB), jnp.int32),
        ),
    )(xf, Wg, bg.reshape(E, 1))


@functools.lru_cache(maxsize=None)
def _sc_dispatch():
    mesh = plsc.VectorSubcoreMesh(core_axis_name="c", subcore_axis_name="s")

    @functools.partial(
        pl.kernel,
        mesh=mesh,
        out_type=jax.ShapeDtypeStruct((NPAD, D), jnp.float32),
        scratch_types=[
            pltpu.VMEM((GCH,), jnp.int32),
            pltpu.VMEM((GCH, D), jnp.float32),
            pltpu.SemaphoreType.DMA,
        ],
    )
    def dispatch(x_hbm, tok_hbm, out_hbm, idx_v, rows_v, sem):
        wid = lax.axis_index("s") * 2 + lax.axis_index("c")
        rows_per_w = NPAD // NW
        for c in range(rows_per_w // GCH):
            base = wid * rows_per_w + c * GCH
            pltpu.sync_copy(tok_hbm.at[pl.ds(base, GCH)], idx_v)
            pltpu.async_copy(x_hbm.at[idx_v], rows_v, sem).wait()
            pltpu.sync_copy(rows_v, out_hbm.at[pl.ds(base, GCH)])

    return dispatch


_SQRT_HALF = 0.7071067811865476


def _gemm_body(be_ref, x_ref, w1_ref, b1_ref, w2_ref, b2_ref, g_ref, o_ref):
    xb = x_ref[...]
    h = jnp.dot(xb, w1_ref[0], preferred_element_type=jnp.float32) + b1_ref[0]
    h = 0.5 * h * (1.0 + lax.erf(h * _SQRT_HALF))
    o = jnp.dot(h, w2_ref[0], preferred_element_type=jnp.float32) + b2_ref[0]
    o_ref[...] = o * g_ref[0, 0][:, None]


def _gemm_call(blk_expert, x_sorted, W1, b1, W2, b2, gate_sorted):
    grid_spec = pltpu.PrefetchScalarGridSpec(
        num_scalar_prefetch=1,
        grid=(NB,),
        in_specs=[
            pl.BlockSpec((BLK, D), lambda i, s: (i, 0)),
            pl.BlockSpec((1, D, F), lambda i, s: (s[i], 0, 0)),
            pl.BlockSpec((1, 1, F), lambda i, s: (s[i], 0, 0)),
            pl.BlockSpec((1, F, D), lambda i, s: (s[i], 0, 0)),
            pl.BlockSpec((1, 1, D), lambda i, s: (s[i], 0, 0)),
            pl.BlockSpec((1, 1, BLK), lambda i, s: (i, 0, 0)),
        ],
        out_specs=pl.BlockSpec((BLK, D), lambda i, s: (i, 0)),
    )
    return pl.pallas_call(
        _gemm_body,
        grid_spec=grid_spec,
        out_shape=jax.ShapeDtypeStruct((NPAD, D), jnp.float32),
    )(blk_expert, x_sorted, W1, b1.reshape(E, 1, F), W2, b2.reshape(E, 1, D),
      gate_sorted.reshape(NB, 1, BLK))


@functools.lru_cache(maxsize=None)
def _sc_combine():
    mesh = plsc.VectorSubcoreMesh(core_axis_name="c", subcore_axis_name="s")

    @functools.partial(
        pl.kernel,
        mesh=mesh,
        out_type=jax.ShapeDtypeStruct((T, D), jnp.float32),
        scratch_types=[
            pltpu.VMEM((CCH,), jnp.int32),
            pltpu.VMEM((CCH, D), jnp.float32),
            pltpu.VMEM((CCH, D), jnp.float32),
            pltpu.SemaphoreType.DMA,
            pltpu.SemaphoreType.DMA,
        ],
    )
    def combine(h_hbm, pos1_hbm, pos2_hbm, out_hbm, idx_v, a_v, b_v, s1, s2):
        wid = lax.axis_index("s") * 2 + lax.axis_index("c")
        base = wid * CCH
        pltpu.sync_copy(pos1_hbm.at[pl.ds(base, CCH)], idx_v)
        pltpu.async_copy(h_hbm.at[idx_v], a_v, s1).wait()
        pltpu.sync_copy(pos2_hbm.at[pl.ds(base, CCH)], idx_v)
        pltpu.async_copy(h_hbm.at[idx_v], b_v, s2).wait()

        def row_add(i, carry):
            for k in range(D // 16):
                sl = pl.ds(k * 16, 16)
                a_v[i, sl] = a_v[i, sl] + b_v[i, sl]
            return carry

        lax.fori_loop(0, CCH, row_add, 0)
        pltpu.sync_copy(a_v, out_hbm.at[pl.ds(base, CCH)])

    return combine


def kernel(x, Wg, bg, W1, b1, W2, b2):
    b, t, d = x.shape
    xf = x.reshape(T, D)
    pos, tok_sorted, gate_sorted, blk_expert = _router_call(xf, Wg, bg)
    x_sorted = _sc_dispatch()(xf, tok_sorted.reshape(NPAD))
    H = _gemm_call(blk_expert.reshape(NB), x_sorted, W1, b1, W2, b2,
                   gate_sorted.reshape(NPAD))
    pos2d = pos.reshape(K, T)
    out = _sc_combine()(H, pos2d[0], pos2d[1])
    return out.reshape(b, t, d)
